# baseline (speedup 1.0000x reference)
import functools

import jax
import jax.numpy as jnp
from jax import lax
from jax.experimental import pallas as pl
from jax.experimental.pallas import tpu as pltpu
from jax.experimental.pallas import tpu_sc as plsc

N_NODES = 10000
N_EDGES = 320000
D = 128

NC = 2
NS = 16
E_TILE = N_EDGES // (NC * NS)
CHUNK = 80
NCHUNK = E_TILE // CHUNK
N_PAD = 10112
RPT = N_PAD // NS
ZC = 79

_MESH = plsc.VectorSubcoreMesh(core_axis_name="c", subcore_axis_name="s")


@functools.partial(
    pl.kernel,
    out_type=jax.ShapeDtypeStruct((NC, N_PAD, 16), jnp.float32),
    mesh=_MESH,
    scratch_types=[
        pltpu.VMEM_SHARED((N_PAD, 16), jnp.float32),
        pltpu.VMEM((NCHUNK, CHUNK), jnp.int32),
        pltpu.VMEM((RPT, 16), jnp.float32),
    ],
)
def _deg_sc(dst_hbm, out_hbm, hist_sh, idx_v, stage_v):
    c = lax.axis_index("c")
    s = lax.axis_index("s")

    def zrow(i, carry):
        stage_v[i] = jnp.zeros((16,), jnp.float32)
        return carry

    lax.fori_loop(0, RPT, zrow, 0)
    pltpu.sync_copy(stage_v, hist_sh.at[pl.ds(s * RPT, RPT)])
    pltpu.sync_copy(dst_hbm.at[c, s], idx_v)

    def orow(i, carry):
        stage_v[i] = jnp.ones((16,), jnp.float32)
        return carry

    lax.fori_loop(0, CHUNK, orow, 0)
    plsc.subcore_barrier()

    def body(j, carry):
        pltpu.sync_copy(
            stage_v.at[pl.ds(0, CHUNK)], hist_sh.at[idx_v.at[j]], add=True
        )
        return carry

    lax.fori_loop(0, NCHUNK, body, 0)
    plsc.subcore_barrier()
    pltpu.sync_copy(
        hist_sh.at[pl.ds(s * RPT, RPT)], out_hbm.at[c, pl.ds(s * RPT, RPT)]
    )


@functools.partial(
    pl.kernel,
    out_type=jax.ShapeDtypeStruct((NC, N_PAD, D), jnp.float32),
    mesh=_MESH,
    scratch_types=[
        pltpu.VMEM_SHARED((N_PAD, D), jnp.float32),
        pltpu.VMEM((E_TILE,), jnp.int32),
        pltpu.VMEM((NCHUNK, CHUNK), jnp.int32),
        pltpu.VMEM((CHUNK, D), jnp.float32),
        pltpu.VMEM((CHUNK, D), jnp.float32),
        pltpu.SemaphoreType.DMA,
        pltpu.SemaphoreType.DMA,
    ],
)
def _edge_sc(g_hbm, src_hbm, dst_hbm, out_hbm, acc_sh, sidx_v, didx_v,
             rows0_v, rows1_v, sem0, sem1):
    c = lax.axis_index("c")
    s = lax.axis_index("s")

    def zrow(j, carry):
        rows0_v[j // 8, pl.ds(16 * (j % 8), 16)] = jnp.zeros(
            (16,), jnp.float32
        )
        return carry

    lax.fori_loop(0, CHUNK * 8, zrow, 0)

    def zcopy(r, carry):
        pltpu.sync_copy(
            rows0_v.at[pl.ds(0, ZC)], acc_sh.at[pl.ds(s * RPT + r * ZC, ZC)]
        )
        return carry

    lax.fori_loop(0, RPT // ZC, zcopy, 0)

    w = c * NS + s
    pltpu.sync_copy(src_hbm.at[pl.ds(w * E_TILE, E_TILE)], sidx_v)
    pltpu.sync_copy(dst_hbm.at[c, s], didx_v)
    plsc.subcore_barrier()

    def start(j, rows, sem):
        pltpu.async_copy(
            g_hbm.at[sidx_v.at[pl.ds(j * CHUNK, CHUNK)]], rows, sem
        )

    def finish(j, rows, sem):
        pltpu.make_async_copy(
            g_hbm.at[sidx_v.at[pl.ds(j * CHUNK, CHUNK)]], rows, sem
        ).wait()
        pltpu.sync_copy(rows, acc_sh.at[didx_v.at[j]], add=True)

    start(0, rows0_v, sem0)

    def body(i, carry):
        j = 2 * i

        @pl.when(j + 1 < NCHUNK)
        def _():
            start(j + 1, rows1_v, sem1)

        finish(j, rows0_v, sem0)

        @pl.when(j + 2 < NCHUNK)
        def _():
            start(j + 2, rows0_v, sem0)

        @pl.when(j + 1 < NCHUNK)
        def _():
            finish(j + 1, rows1_v, sem1)

        return carry

    lax.fori_loop(0, (NCHUNK + 1) // 2, body, 0)
    plsc.subcore_barrier()
    pltpu.sync_copy(
        acc_sh.at[pl.ds(s * RPT, RPT)], out_hbm.at[c, pl.ds(s * RPT, RPT)]
    )


_BLK = 2000
_NBLK = N_NODES // _BLK


def _dinv_of(degp_ref):
    deg = degp_ref[0][:, 0:1] + degp_ref[1][:, 0:1] + 1.0
    return lax.rsqrt(deg)


def _tca_body(x_ref, w_ref, degp_ref, g_ref):
    dinv = _dinv_of(degp_ref)
    g_ref[...] = (
        jnp.dot(x_ref[...], w_ref[...], preferred_element_type=jnp.float32)
        * dinv
    )


def _tcb_body(sp_ref, g1_ref, degp_ref, b_ref, w_ref, g2_ref):
    dinv = _dinv_of(degp_ref)
    t = (sp_ref[0] + sp_ref[1] + g1_ref[...]) * dinv + b_ref[...]
    h2 = jnp.maximum(t, 0.0)
    g2_ref[...] = (
        jnp.dot(h2, w_ref[...], preferred_element_type=jnp.float32) * dinv
    )


def _tcc_body(sp_ref, g2_ref, degp_ref, b_ref, out_ref):
    dinv = _dinv_of(degp_ref)
    out_ref[...] = (sp_ref[0] + sp_ref[1] + g2_ref[...]) * dinv + b_ref[...]


_ROWS_SPEC = pl.BlockSpec((_BLK, D), lambda i: (i, 0))
_MAT_SPEC = pl.BlockSpec((D, D), lambda i: (0, 0))
_DEGP_SPEC = pl.BlockSpec((2, _BLK, 16), lambda i: (0, i, 0))
_PART_SPEC = pl.BlockSpec((2, _BLK, D), lambda i: (0, i, 0))
_BIAS_SPEC = pl.BlockSpec((1, D), lambda i: (0, 0))
_ROWS_OUT = jax.ShapeDtypeStruct((N_NODES, D), jnp.float32)

_tca = pl.pallas_call(
    _tca_body,
    grid=(_NBLK,),
    in_specs=[_ROWS_SPEC, _MAT_SPEC, _DEGP_SPEC],
    out_specs=_ROWS_SPEC,
    out_shape=_ROWS_OUT,
)

_tcb = pl.pallas_call(
    _tcb_body,
    grid=(_NBLK,),
    in_specs=[_PART_SPEC, _ROWS_SPEC, _DEGP_SPEC, _BIAS_SPEC, _MAT_SPEC],
    out_specs=_ROWS_SPEC,
    out_shape=_ROWS_OUT,
)

_tcc = pl.pallas_call(
    _tcc_body,
    grid=(_NBLK,),
    in_specs=[_PART_SPEC, _ROWS_SPEC, _DEGP_SPEC, _BIAS_SPEC],
    out_specs=_ROWS_SPEC,
    out_shape=_ROWS_OUT,
)


def kernel(x, edge_index, W1, b1, W2, b2):
    ei = edge_index.astype(jnp.int32)
    src = ei[0]
    dst = ei[1].reshape(NC, NS, NCHUNK, CHUNK)
    b1r = b1.reshape(1, D)
    b2r = b2.reshape(1, D)

    degp = _deg_sc(dst)
    g1 = _tca(x, W1, degp)
    s1 = _edge_sc(g1, src, dst)
    g2 = _tcb(s1, g1, degp, b1r, W2)
    s2 = _edge_sc(g2, src, dst)
    out = _tcc(s2, g2, degp, b2r)
    return out


# speedup vs baseline: 1.0163x; 1.0163x over previous
import functools

import jax
import jax.numpy as jnp
from jax import lax
from jax.experimental import pallas as pl
from jax.experimental.pallas import tpu as pltpu
from jax.experimental.pallas import tpu_sc as plsc

N_NODES = 10000
N_EDGES = 320000
D = 128

NC = 2
NS = 16
E_TILE = N_EDGES // (NC * NS)
CHUNK = 80
NCHUNK = E_TILE // CHUNK
N_PAD = 10112
RPT = N_PAD // NS
ZC = 79

_MESH = plsc.VectorSubcoreMesh(core_axis_name="c", subcore_axis_name="s")


@functools.partial(
    pl.kernel,
    out_type=jax.ShapeDtypeStruct((NC, N_PAD, 16), jnp.float32),
    mesh=_MESH,
    scratch_types=[
        pltpu.VMEM_SHARED((N_PAD, 16), jnp.float32),
        pltpu.VMEM((NCHUNK, CHUNK), jnp.int32),
        pltpu.VMEM((RPT, 16), jnp.float32),
        pltpu.SemaphoreType.DMA,
    ],
)
def _deg_sc(dst_hbm, out_hbm, hist_sh, idx_v, stage_v, sem):
    c = lax.axis_index("c")
    s = lax.axis_index("s")

    def zrow(i, carry):
        stage_v[i] = jnp.zeros((16,), jnp.float32)
        return carry

    lax.fori_loop(0, RPT, zrow, 0)
    pltpu.sync_copy(stage_v, hist_sh.at[pl.ds(s * RPT, RPT)])
    pltpu.sync_copy(dst_hbm.at[c, s], idx_v)

    def orow(i, carry):
        stage_v[i] = jnp.ones((16,), jnp.float32)
        return carry

    lax.fori_loop(0, CHUNK, orow, 0)
    plsc.subcore_barrier()

    def body(j, carry):
        pltpu.sync_copy(
            stage_v.at[pl.ds(0, CHUNK)], hist_sh.at[idx_v.at[j]], add=True
        )
        return carry

    lax.fori_loop(0, NCHUNK, body, 0)
    plsc.subcore_barrier()
    pltpu.sync_copy(
        hist_sh.at[pl.ds(s * RPT, RPT)], out_hbm.at[c, pl.ds(s * RPT, RPT)]
    )


@functools.partial(
    pl.kernel,
    out_type=jax.ShapeDtypeStruct((NC, N_PAD, D), jnp.float32),
    mesh=_MESH,
    scratch_types=[
        pltpu.VMEM_SHARED((N_PAD, D), jnp.float32),
        pltpu.VMEM((E_TILE,), jnp.int32),
        pltpu.VMEM((NCHUNK, CHUNK), jnp.int32),
        pltpu.VMEM((CHUNK, D), jnp.float32),
        pltpu.VMEM((CHUNK, D), jnp.float32),
        pltpu.SemaphoreType.DMA,
        pltpu.SemaphoreType.DMA,
    ],
)
def _edge_sc(g_hbm, src_hbm, dst_hbm, out_hbm, acc_sh, sidx_v, didx_v,
             rows0_v, rows1_v, sem0, sem1):
    c = lax.axis_index("c")
    s = lax.axis_index("s")

    w = c * NS + s
    pltpu.async_copy(src_hbm.at[pl.ds(w * E_TILE, E_TILE)], sidx_v, sem0)
    pltpu.async_copy(dst_hbm.at[c, s], didx_v, sem1)

    def zrow(j, carry):
        rows0_v[j // 8, pl.ds(16 * (j % 8), 16)] = jnp.zeros(
            (16,), jnp.float32
        )
        return carry

    lax.fori_loop(0, CHUNK * 8, zrow, 0)

    def zcopy(r, carry):
        pltpu.sync_copy(
            rows0_v.at[pl.ds(0, ZC)], acc_sh.at[pl.ds(s * RPT + r * ZC, ZC)]
        )
        return carry

    lax.fori_loop(0, RPT // ZC, zcopy, 0)

    pltpu.make_async_copy(
        src_hbm.at[pl.ds(w * E_TILE, E_TILE)], sidx_v, sem0
    ).wait()
    pltpu.make_async_copy(dst_hbm.at[c, s], didx_v, sem1).wait()
    plsc.subcore_barrier()

    def start(j, rows, sem):
        pltpu.async_copy(
            g_hbm.at[sidx_v.at[pl.ds(j * CHUNK, CHUNK)]], rows, sem
        )

    def finish(j, rows, sem):
        pltpu.make_async_copy(
            g_hbm.at[sidx_v.at[pl.ds(j * CHUNK, CHUNK)]], rows, sem
        ).wait()
        pltpu.sync_copy(rows, acc_sh.at[didx_v.at[j]], add=True)

    start(0, rows0_v, sem0)

    def body(i, carry):
        j = 2 * i

        @pl.when(j + 1 < NCHUNK)
        def _():
            start(j + 1, rows1_v, sem1)

        finish(j, rows0_v, sem0)

        @pl.when(j + 2 < NCHUNK)
        def _():
            start(j + 2, rows0_v, sem0)

        @pl.when(j + 1 < NCHUNK)
        def _():
            finish(j + 1, rows1_v, sem1)

        return carry

    lax.fori_loop(0, (NCHUNK + 1) // 2, body, 0)
    plsc.subcore_barrier()
    pltpu.sync_copy(
        acc_sh.at[pl.ds(s * RPT, RPT)], out_hbm.at[c, pl.ds(s * RPT, RPT)]
    )


_BLK = 2000
_NBLK = N_NODES // _BLK


def _dinv_of(degp_ref):
    deg = degp_ref[0][:, 0:1] + degp_ref[1][:, 0:1] + 1.0
    return lax.rsqrt(deg)


def _tca_body(x_ref, w_ref, degp_ref, g_ref):
    dinv = _dinv_of(degp_ref)
    g_ref[...] = (
        jnp.dot(x_ref[...], w_ref[...], preferred_element_type=jnp.float32)
        * dinv
    )


def _tcb_body(sp_ref, g1_ref, degp_ref, b_ref, w_ref, g2_ref):
    dinv = _dinv_of(degp_ref)
    t = (sp_ref[0] + sp_ref[1] + g1_ref[...]) * dinv + b_ref[...]
    h2 = jnp.maximum(t, 0.0)
    g2_ref[...] = (
        jnp.dot(h2, w_ref[...], preferred_element_type=jnp.float32) * dinv
    )


def _tcc_body(sp_ref, g2_ref, degp_ref, b_ref, out_ref):
    dinv = _dinv_of(degp_ref)
    out_ref[...] = (sp_ref[0] + sp_ref[1] + g2_ref[...]) * dinv + b_ref[...]


_ROWS_SPEC = pl.BlockSpec((_BLK, D), lambda i: (i, 0))
_MAT_SPEC = pl.BlockSpec((D, D), lambda i: (0, 0))
_DEGP_SPEC = pl.BlockSpec((2, _BLK, 16), lambda i: (0, i, 0))
_PART_SPEC = pl.BlockSpec((2, _BLK, D), lambda i: (0, i, 0))
_BIAS_SPEC = pl.BlockSpec((1, D), lambda i: (0, 0))
_ROWS_OUT = jax.ShapeDtypeStruct((N_NODES, D), jnp.float32)

_tca = pl.pallas_call(
    _tca_body,
    grid=(_NBLK,),
    in_specs=[_ROWS_SPEC, _MAT_SPEC, _DEGP_SPEC],
    out_specs=_ROWS_SPEC,
    out_shape=_ROWS_OUT,
)

_tcb = pl.pallas_call(
    _tcb_body,
    grid=(_NBLK,),
    in_specs=[_PART_SPEC, _ROWS_SPEC, _DEGP_SPEC, _BIAS_SPEC, _MAT_SPEC],
    out_specs=_ROWS_SPEC,
    out_shape=_ROWS_OUT,
)

_tcc = pl.pallas_call(
    _tcc_body,
    grid=(_NBLK,),
    in_specs=[_PART_SPEC, _ROWS_SPEC, _DEGP_SPEC, _BIAS_SPEC],
    out_specs=_ROWS_SPEC,
    out_shape=_ROWS_OUT,
)


def kernel(x, edge_index, W1, b1, W2, b2):
    ei = edge_index.astype(jnp.int32)
    src = ei[0]
    dst = ei[1].reshape(NC, NS, NCHUNK, CHUNK)
    b1r = b1.reshape(1, D)
    b2r = b2.reshape(1, D)

    degp = _deg_sc(dst)
    g1 = _tca(x, W1, degp)
    s1 = _edge_sc(g1, src, dst)
    g2 = _tcb(s1, g1, degp, b1r, W2)
    s2 = _edge_sc(g2, src, dst)
    out = _tcc(s2, g2, degp, b2r)
    return out
